# baseline (speedup 1.0000x reference)
import jax
import jax.numpy as jnp
from jax.experimental import pallas as pl
from jax.experimental.pallas import tpu as pltpu

_SPLITS = (5000, 3000, 2000)
_OFFS = (0, 5000, 8000)


def _body(xu, xi, ou, oi, vu, vi, sin_u, sin_i, sout_u, sout_i):
    nck = len(_SPLITS)
    for k in range(nck):
        sl = pl.ds(_OFFS[k], _SPLITS[k])
        pltpu.make_async_copy(xu.at[sl], vu.at[sl], sin_u.at[k]).start()
    for k in range(nck):
        sl = pl.ds(_OFFS[k], _SPLITS[k])
        pltpu.make_async_copy(xi.at[sl], vi.at[sl], sin_i.at[k]).start()
    for k in range(nck):
        sl = pl.ds(_OFFS[k], _SPLITS[k])
        pltpu.make_async_copy(xu.at[sl], vu.at[sl], sin_u.at[k]).wait()
        pltpu.make_async_copy(vu.at[sl], ou.at[sl], sout_u.at[k]).start()
    for k in range(nck):
        sl = pl.ds(_OFFS[k], _SPLITS[k])
        pltpu.make_async_copy(xi.at[sl], vi.at[sl], sin_i.at[k]).wait()
        pltpu.make_async_copy(vi.at[sl], oi.at[sl], sout_i.at[k]).start()
    for k in range(nck):
        sl = pl.ds(_OFFS[k], _SPLITS[k])
        pltpu.make_async_copy(vu.at[sl], ou.at[sl], sout_u.at[k]).wait()
        pltpu.make_async_copy(vi.at[sl], oi.at[sl], sout_i.at[k]).wait()


def kernel(x_user, x_item, edge_index):
    del edge_index
    n, d = x_user.shape
    nck = len(_SPLITS)
    ou, oi = pl.pallas_call(
        _body,
        in_specs=[
            pl.BlockSpec(memory_space=pl.ANY),
            pl.BlockSpec(memory_space=pl.ANY),
        ],
        out_specs=[
            pl.BlockSpec(memory_space=pl.ANY),
            pl.BlockSpec(memory_space=pl.ANY),
        ],
        out_shape=[
            jax.ShapeDtypeStruct((n, d), x_user.dtype),
            jax.ShapeDtypeStruct((n, d), x_item.dtype),
        ],
        scratch_shapes=[
            pltpu.VMEM((n, d), jnp.float32),
            pltpu.VMEM((n, d), jnp.float32),
            pltpu.SemaphoreType.DMA((nck,)),
            pltpu.SemaphoreType.DMA((nck,)),
            pltpu.SemaphoreType.DMA((nck,)),
            pltpu.SemaphoreType.DMA((nck,)),
        ],
    )(x_user, x_item)
    return (ou, oi)


# speedup vs baseline: 1.0394x; 1.0394x over previous
import jax
import jax.numpy as jnp
from jax.experimental import pallas as pl
from jax.experimental.pallas import tpu as pltpu


def _body(splits, xu, xi, ou, oi, vu, vi, sin_u, sin_i, sout_u, sout_i):
    nck = len(splits)
    for k in range(nck):
        sl = pl.ds(*splits[k])
        pltpu.make_async_copy(xu.at[sl], vu.at[sl], sin_u.at[k]).start()
    for k in range(nck):
        sl = pl.ds(*splits[k])
        pltpu.make_async_copy(xi.at[sl], vi.at[sl], sin_i.at[k]).start()
    for k in range(nck):
        sl = pl.ds(*splits[k])
        pltpu.make_async_copy(xu.at[sl], vu.at[sl], sin_u.at[k]).wait()
        pltpu.make_async_copy(vu.at[sl], ou.at[sl], sout_u.at[k]).start()
    for k in range(nck):
        sl = pl.ds(*splits[k])
        pltpu.make_async_copy(xi.at[sl], vi.at[sl], sin_i.at[k]).wait()
        pltpu.make_async_copy(vi.at[sl], oi.at[sl], sout_i.at[k]).start()
    for k in range(nck):
        sl = pl.ds(*splits[k])
        pltpu.make_async_copy(vu.at[sl], ou.at[sl], sout_u.at[k]).wait()
        pltpu.make_async_copy(vi.at[sl], oi.at[sl], sout_i.at[k]).wait()


def kernel(x_user, x_item, edge_index):
    del edge_index
    n, d = x_user.shape
    half = (n // 2) & ~7
    splits = ((0, half), (half, n - half))
    nck = len(splits)

    def body(*refs):
        return _body(splits, *refs)

    ou, oi = pl.pallas_call(
        body,
        in_specs=[
            pl.BlockSpec(memory_space=pl.ANY),
            pl.BlockSpec(memory_space=pl.ANY),
        ],
        out_specs=[
            pl.BlockSpec(memory_space=pl.ANY),
            pl.BlockSpec(memory_space=pl.ANY),
        ],
        out_shape=[
            jax.ShapeDtypeStruct((n, d), x_user.dtype),
            jax.ShapeDtypeStruct((n, d), x_item.dtype),
        ],
        scratch_shapes=[
            pltpu.VMEM((n, d), jnp.float32),
            pltpu.VMEM((n, d), jnp.float32),
            pltpu.SemaphoreType.DMA((nck,)),
            pltpu.SemaphoreType.DMA((nck,)),
            pltpu.SemaphoreType.DMA((nck,)),
            pltpu.SemaphoreType.DMA((nck,)),
        ],
    )(x_user, x_item)
    return (ou, oi)


# speedup vs baseline: 1.0489x; 1.0090x over previous
import jax
import jax.numpy as jnp
from jax.experimental import pallas as pl
from jax.experimental.pallas import tpu as pltpu


def _body(splits, xu, xi, ou, oi, vu, vi, sin_u, sin_i, sout_u, sout_i):
    nck = len(splits)
    for k in range(nck):
        sl = pl.ds(*splits[k])
        pltpu.make_async_copy(xu.at[sl], vu.at[sl], sin_u.at[k]).start()
    for k in range(nck):
        sl = pl.ds(*splits[k])
        pltpu.make_async_copy(xi.at[sl], vi.at[sl], sin_i.at[k]).start()
    for k in range(nck):
        sl = pl.ds(*splits[k])
        pltpu.make_async_copy(xu.at[sl], vu.at[sl], sin_u.at[k]).wait()
        pltpu.make_async_copy(vu.at[sl], ou.at[sl], sout_u.at[k]).start()
    for k in range(nck):
        sl = pl.ds(*splits[k])
        pltpu.make_async_copy(xi.at[sl], vi.at[sl], sin_i.at[k]).wait()
        pltpu.make_async_copy(vi.at[sl], oi.at[sl], sout_i.at[k]).start()
    for k in range(nck):
        sl = pl.ds(*splits[k])
        pltpu.make_async_copy(vu.at[sl], ou.at[sl], sout_u.at[k]).wait()
        pltpu.make_async_copy(vi.at[sl], oi.at[sl], sout_i.at[k]).wait()


def kernel(x_user, x_item, edge_index):
    del edge_index
    n, d = x_user.shape
    half = (n // 2) & ~7
    splits = ((0, half), (half, n - half))
    nck = len(splits)

    def body(*refs):
        return _body(splits, *refs)

    ou, oi = pl.pallas_call(
        body,
        in_specs=[
            pl.BlockSpec(memory_space=pl.ANY),
            pl.BlockSpec(memory_space=pl.ANY),
        ],
        out_specs=[
            pl.BlockSpec(memory_space=pl.ANY),
            pl.BlockSpec(memory_space=pl.ANY),
        ],
        out_shape=[
            jax.ShapeDtypeStruct((n, d), x_user.dtype),
            jax.ShapeDtypeStruct((n, d), x_item.dtype),
        ],
        scratch_shapes=[
            pltpu.VMEM((n, d), x_user.dtype),
            pltpu.VMEM((n, d), x_item.dtype),
            pltpu.SemaphoreType.DMA((nck,)),
            pltpu.SemaphoreType.DMA((nck,)),
            pltpu.SemaphoreType.DMA((nck,)),
            pltpu.SemaphoreType.DMA((nck,)),
        ],
    )(x_user, x_item)
    return (ou, oi)
